# baseline (speedup 1.0000x reference)
import functools

import jax
import jax.numpy as jnp
from jax import lax
from jax.experimental import pallas as pl
from jax.experimental.pallas import tpu as pltpu
from jax.experimental.pallas import tpu_sc as plsc

ALPHA = 0.5
NC = 2
NS = 16
L = 16


def _hist_body(edges_hbm, zeros_hbm, out_hbm,
               local, idxbuf, tmp, accb, shared, n_pad, ept):
    c = lax.axis_index("c")
    s = lax.axis_index("s")
    e = ept * NS
    seg = n_pad // NS
    pltpu.sync_copy(edges_hbm.at[pl.ds(c * e + s * ept, ept)], idxbuf)
    pltpu.sync_copy(zeros_hbm, local)

    ones = jnp.full((L,), 1.0, jnp.float32)

    def body(i, carry):
        idxv = idxbuf[pl.ds(i * L, L)]
        plsc.addupdate_scatter(local, [idxv], ones)
        return carry

    lax.fori_loop(0, ept // L, body, 0)

    pltpu.sync_copy(local, shared.at[pl.ds(s * n_pad, n_pad)])
    plsc.subcore_barrier()

    pltpu.sync_copy(zeros_hbm.at[pl.ds(0, seg)], accb)
    for k in range(NS):
        pltpu.sync_copy(shared.at[pl.ds(k * n_pad + s * seg, seg)], tmp)

        def addk(j, carry):
            sl = pl.ds(j * L, L)
            accb[sl] = accb[sl] + tmp[sl]
            return carry

        lax.fori_loop(0, seg // L, addk, 0)

    pltpu.sync_copy(accb, out_hbm.at[pl.ds(c * n_pad + s * seg, seg)])


def _make_hist(n_pad, e):
    ept = e // NS
    seg = n_pad // NS
    return pl.kernel(
        functools.partial(_hist_body, n_pad=n_pad, ept=ept),
        out_type=jax.ShapeDtypeStruct((NC * n_pad,), jnp.float32),
        mesh=plsc.VectorSubcoreMesh(
            core_axis_name="c", subcore_axis_name="s",
            num_cores=NC, num_subcores=NS),
        compiler_params=pltpu.CompilerParams(needs_layout_passes=False),
        scratch_types=[
            pltpu.VMEM((n_pad,), jnp.float32),
            pltpu.VMEM((e // NS,), jnp.int32),
            pltpu.VMEM((seg,), jnp.float32),
            pltpu.VMEM((seg,), jnp.float32),
            pltpu.VMEM_SHARED((NS * n_pad,), jnp.float32),
        ],
    )


_C = 128
_NQ = 4


def _scatter_body(gidx_hbm, sidx_hbm, feat_hbm, zeros_hbm, out_hbm,
                  g0, s0, buf0, sem0, acc,
                  n_pad, chunks):
    c = lax.axis_index("c")
    s = lax.axis_index("s")
    rpt = n_pad // NS
    base = (c * NS + s) * (chunks + 1) * _C
    pltpu.sync_copy(zeros_hbm, acc.at[pl.ds(s * rpt, rpt)])
    plsc.subcore_barrier()

    def load_idx(j, gv, sv):
        pltpu.sync_copy(gidx_hbm.at[pl.ds(base + j * _C, _C)], gv)
        pltpu.sync_copy(sidx_hbm.at[pl.ds(base + j * _C, _C)], sv)

    def body(k, carry):
        load_idx(k, g0, s0)
        pltpu.async_copy(feat_hbm.at[g0], buf0, sem0).wait()
        pltpu.sync_copy(buf0, acc.at[s0], add=True)
        return carry

    lax.fori_loop(0, chunks, body, 0)

    plsc.subcore_barrier()
    pltpu.sync_copy(acc.at[pl.ds(s * rpt, rpt)],
                    out_hbm.at[c, pl.ds(s * rpt, rpt)])


def _make_scatter(n_pad, d, chunks):
    rpt = n_pad // NS
    qc = chunks // _NQ
    return pl.kernel(
        functools.partial(_scatter_body, n_pad=n_pad, chunks=chunks),
        out_type=jax.ShapeDtypeStruct((NC, n_pad, d), jnp.float32),
        mesh=plsc.VectorSubcoreMesh(
            core_axis_name="c", subcore_axis_name="s",
            num_cores=NC, num_subcores=NS),
        compiler_params=pltpu.CompilerParams(needs_layout_passes=False),
        scratch_types=[
            pltpu.VMEM((_C,), jnp.int32),
            pltpu.VMEM((_C,), jnp.int32),
            pltpu.VMEM((_C, d), jnp.float32),
            pltpu.SemaphoreType.DMA,
            pltpu.VMEM_SHARED((n_pad, d), jnp.float32),
        ],
    )


def _inv_block(deg):
    return jnp.where(deg > 0.0, lax.rsqrt(jnp.maximum(deg, 1e-30)), 0.0)


def _tc_transform_body(x_ref, ws_ref, wd_ref, od_ref, id_ref, out_ref):
    xb = x_ref[...]
    ii = _inv_block(id_ref[...])
    oi = _inv_block(od_ref[...])
    dn = (((1,), (1,)), ((), ()))
    fs = ALPHA * lax.dot_general(xb, ws_ref[...], dn,
                                 preferred_element_type=jnp.float32,
                                 precision=lax.Precision.HIGHEST)
    fd = (1.0 - ALPHA) * lax.dot_general(xb, wd_ref[...], dn,
                                         preferred_element_type=jnp.float32,
                                         precision=lax.Precision.HIGHEST)
    out_ref[0] = ii * fs
    out_ref[1] = oi * fd


def _tc_combine_body(p_ref, od_ref, id_ref, bs_ref, bd_ref, out_ref):
    oi = _inv_block(od_ref[...])
    ii = _inv_block(id_ref[...])
    bias = ALPHA * bs_ref[...] + (1.0 - ALPHA) * bd_ref[...]
    out_ref[...] = oi * p_ref[0] + ii * p_ref[1] + bias


def kernel(x, edge_index, W_src, b_src, W_dst, b_dst):
    n, d_in = x.shape
    d_out = W_src.shape[0]
    e = edge_index.shape[1]
    n_pad = -(-n // (NS * L)) * (NS * L)
    assert e % (NS * L) == 0

    row = edge_index[0]
    col = edge_index[1]

    zeros_h = jnp.zeros((n_pad,), jnp.float32)
    deg = _make_hist(n_pad, e)(edge_index.reshape(2 * e), zeros_h)
    od = deg[0:n].reshape(n, 1)
    idg = deg[n_pad:n_pad + n].reshape(n, 1)

    B = 2000
    feat = pl.pallas_call(
        _tc_transform_body,
        grid=(n // B,),
        in_specs=[
            pl.BlockSpec((B, d_in), lambda i: (i, 0)),
            pl.BlockSpec((d_out, d_in), lambda i: (0, 0)),
            pl.BlockSpec((d_out, d_in), lambda i: (0, 0)),
            pl.BlockSpec((B, 1), lambda i: (i, 0)),
            pl.BlockSpec((B, 1), lambda i: (i, 0)),
        ],
        out_specs=pl.BlockSpec((2, B, d_out), lambda i: (0, i, 0)),
        out_shape=jax.ShapeDtypeStruct((2, n, d_out), jnp.float32),
    )(x, W_src, W_dst, od, idg)
    feat_flat = feat.reshape(2 * n, d_out)

    chunks = -(-e // (NS * _C * 2 * _NQ)) * (2 * _NQ)
    ep = chunks * _C * NS
    pad = ep - e
    pad_g = jnp.zeros((pad,), jnp.int32)
    pad_s = jnp.full((pad,), n, jnp.int32)
    gidx = jnp.stack([jnp.concatenate([col, pad_g]),
                      jnp.concatenate([row + n, pad_g])])
    sidx = jnp.stack([jnp.concatenate([row, pad_s]),
                      jnp.concatenate([col, pad_s])])
    tail = ((0, 0), (0, _C))
    gidx = jnp.pad(gidx.reshape(NC * NS, chunks * _C), tail).reshape(-1)
    sidx = jnp.pad(sidx.reshape(NC * NS, chunks * _C), tail).reshape(-1)
    zeros_f = jnp.zeros((n_pad // NS, d_out), jnp.float32)
    part = _make_scatter(n_pad, d_out, chunks)(gidx, sidx, feat_flat, zeros_f)

    out = pl.pallas_call(
        _tc_combine_body,
        grid=(n // B,),
        in_specs=[
            pl.BlockSpec((2, B, d_out), lambda i: (0, i, 0)),
            pl.BlockSpec((B, 1), lambda i: (i, 0)),
            pl.BlockSpec((B, 1), lambda i: (i, 0)),
            pl.BlockSpec((1, d_out), lambda i: (0, 0)),
            pl.BlockSpec((1, d_out), lambda i: (0, 0)),
        ],
        out_specs=pl.BlockSpec((B, d_out), lambda i: (i, 0)),
        out_shape=jax.ShapeDtypeStruct((n, d_out), jnp.float32),
    )(part, od, idg, b_src.reshape(1, d_out), b_dst.reshape(1, d_out))
    return out


# speedup vs baseline: 1.6240x; 1.6240x over previous
import functools

import jax
import jax.numpy as jnp
from jax import lax
from jax.experimental import pallas as pl
from jax.experimental.pallas import tpu as pltpu
from jax.experimental.pallas import tpu_sc as plsc

ALPHA = 0.5
NC = 2
NS = 16
L = 16


def _hist_body(edges_hbm, zeros_hbm, out_hbm,
               local, idxbuf, tmp, accb, shared, n_pad, ept):
    c = lax.axis_index("c")
    s = lax.axis_index("s")
    e = ept * NS
    seg = n_pad // NS
    pltpu.sync_copy(edges_hbm.at[pl.ds(c * e + s * ept, ept)], idxbuf)
    pltpu.sync_copy(zeros_hbm, local)

    ones = jnp.full((L,), 1.0, jnp.float32)

    def body(i, carry):
        idxv = idxbuf[pl.ds(i * L, L)]
        plsc.addupdate_scatter(local, [idxv], ones)
        return carry

    lax.fori_loop(0, ept // L, body, 0)

    pltpu.sync_copy(local, shared.at[pl.ds(s * n_pad, n_pad)])
    plsc.subcore_barrier()

    pltpu.sync_copy(zeros_hbm.at[pl.ds(0, seg)], accb)
    for k in range(NS):
        pltpu.sync_copy(shared.at[pl.ds(k * n_pad + s * seg, seg)], tmp)

        def addk(j, carry):
            sl = pl.ds(j * L, L)
            accb[sl] = accb[sl] + tmp[sl]
            return carry

        lax.fori_loop(0, seg // L, addk, 0)

    pltpu.sync_copy(accb, out_hbm.at[pl.ds(c * n_pad + s * seg, seg)])


def _make_hist(n_pad, e):
    ept = e // NS
    seg = n_pad // NS
    return pl.kernel(
        functools.partial(_hist_body, n_pad=n_pad, ept=ept),
        out_type=jax.ShapeDtypeStruct((NC * n_pad,), jnp.float32),
        mesh=plsc.VectorSubcoreMesh(
            core_axis_name="c", subcore_axis_name="s",
            num_cores=NC, num_subcores=NS),
        compiler_params=pltpu.CompilerParams(needs_layout_passes=False),
        scratch_types=[
            pltpu.VMEM((n_pad,), jnp.float32),
            pltpu.VMEM((e // NS,), jnp.int32),
            pltpu.VMEM((seg,), jnp.float32),
            pltpu.VMEM((seg,), jnp.float32),
            pltpu.VMEM_SHARED((NS * n_pad,), jnp.float32),
        ],
    )


_C = 128
_NQ = 4


def _scatter_body(gidx_hbm, sidx_hbm, feat_hbm, zeros_hbm, out_hbm,
                  g0, s0, buf0, sem0, acc,
                  n_pad, chunks):
    c = lax.axis_index("c")
    s = lax.axis_index("s")
    rpt = n_pad // NS
    base = (c * NS + s) * chunks * _C
    pltpu.sync_copy(zeros_hbm, acc.at[pl.ds(s * rpt, rpt)])
    plsc.subcore_barrier()

    def load_idx(j, gv, sv):
        pltpu.sync_copy(gidx_hbm.at[pl.ds(base + j * _C, _C)], gv)
        pltpu.sync_copy(sidx_hbm.at[pl.ds(base + j * _C, _C)], sv)

    def body(k, carry):
        load_idx(k, g0, s0)
        pltpu.async_copy(feat_hbm.at[g0], buf0, sem0).wait()
        pltpu.sync_copy(buf0, acc.at[s0], add=True)
        return carry

    lax.fori_loop(0, chunks, body, 0)

    plsc.subcore_barrier()
    pltpu.sync_copy(acc.at[pl.ds(s * rpt, rpt)],
                    out_hbm.at[c, pl.ds(s * rpt, rpt)])


def _make_scatter(n_pad, d, chunks):
    rpt = n_pad // NS
    qc = chunks // _NQ
    return pl.kernel(
        functools.partial(_scatter_body, n_pad=n_pad, chunks=chunks),
        out_type=jax.ShapeDtypeStruct((NC, n_pad, d), jnp.float32),
        mesh=plsc.VectorSubcoreMesh(
            core_axis_name="c", subcore_axis_name="s",
            num_cores=NC, num_subcores=NS),
        compiler_params=pltpu.CompilerParams(needs_layout_passes=False),
        scratch_types=[
            pltpu.VMEM((_C,), jnp.int32),
            pltpu.VMEM((_C,), jnp.int32),
            pltpu.VMEM((_C, d), jnp.float32),
            pltpu.SemaphoreType.DMA,
            pltpu.VMEM_SHARED((n_pad, d), jnp.float32),
        ],
    )


def _inv_block(deg):
    return jnp.where(deg > 0.0, lax.rsqrt(jnp.maximum(deg, 1e-30)), 0.0)


def _tc_transform_body(x_ref, ws_ref, wd_ref, od_ref, id_ref, out_ref):
    xb = x_ref[...]
    ii = _inv_block(id_ref[...])
    oi = _inv_block(od_ref[...])
    dn = (((1,), (1,)), ((), ()))
    fs = ALPHA * lax.dot_general(xb, ws_ref[...], dn,
                                 preferred_element_type=jnp.float32,
                                 precision=lax.Precision.HIGHEST)
    fd = (1.0 - ALPHA) * lax.dot_general(xb, wd_ref[...], dn,
                                         preferred_element_type=jnp.float32,
                                         precision=lax.Precision.HIGHEST)
    out_ref[0] = ii * fs
    out_ref[1] = oi * fd


def _tc_combine_body(p_ref, od_ref, id_ref, bs_ref, bd_ref, out_ref):
    oi = _inv_block(od_ref[...])
    ii = _inv_block(id_ref[...])
    bias = ALPHA * bs_ref[...] + (1.0 - ALPHA) * bd_ref[...]
    out_ref[...] = oi * p_ref[0] + ii * p_ref[1] + bias


def kernel(x, edge_index, W_src, b_src, W_dst, b_dst):
    n, d_in = x.shape
    d_out = W_src.shape[0]
    e = edge_index.shape[1]
    n_pad = -(-n // (NS * L)) * (NS * L)
    assert e % (NS * L) == 0

    row = edge_index[0]
    col = edge_index[1]

    zeros_h = jnp.zeros((n_pad,), jnp.float32)
    deg = _make_hist(n_pad, e)(edge_index.reshape(2 * e), zeros_h)
    od = deg[0:n].reshape(n, 1)
    idg = deg[n_pad:n_pad + n].reshape(n, 1)

    B = 2000
    feat = pl.pallas_call(
        _tc_transform_body,
        grid=(n // B,),
        in_specs=[
            pl.BlockSpec((B, d_in), lambda i: (i, 0)),
            pl.BlockSpec((d_out, d_in), lambda i: (0, 0)),
            pl.BlockSpec((d_out, d_in), lambda i: (0, 0)),
            pl.BlockSpec((B, 1), lambda i: (i, 0)),
            pl.BlockSpec((B, 1), lambda i: (i, 0)),
        ],
        out_specs=pl.BlockSpec((2, B, d_out), lambda i: (0, i, 0)),
        out_shape=jax.ShapeDtypeStruct((2, n, d_out), jnp.float32),
    )(x, W_src, W_dst, od, idg)
    feat_flat = feat.reshape(2 * n, d_out)

    chunks = -(-e // (NS * _C))
    ep = chunks * _C * NS
    pad = ep - e
    pad_g = jnp.zeros((pad,), jnp.int32)
    pad_s = jnp.full((pad,), n, jnp.int32)
    gidx = jnp.stack([jnp.concatenate([col, pad_g]),
                      jnp.concatenate([row + n, pad_g])])
    sidx = jnp.stack([jnp.concatenate([row, pad_s]),
                      jnp.concatenate([col, pad_s])])
    gidx = gidx.reshape(NC * NS * chunks * _C)
    sidx = sidx.reshape(NC * NS * chunks * _C)
    zeros_f = jnp.zeros((n_pad // NS, d_out), jnp.float32)
    part = _make_scatter(n_pad, d_out, chunks)(gidx, sidx, feat_flat, zeros_f)

    out = pl.pallas_call(
        _tc_combine_body,
        grid=(n // B,),
        in_specs=[
            pl.BlockSpec((2, B, d_out), lambda i: (0, i, 0)),
            pl.BlockSpec((B, 1), lambda i: (i, 0)),
            pl.BlockSpec((B, 1), lambda i: (i, 0)),
            pl.BlockSpec((1, d_out), lambda i: (0, 0)),
            pl.BlockSpec((1, d_out), lambda i: (0, 0)),
        ],
        out_specs=pl.BlockSpec((B, d_out), lambda i: (i, 0)),
        out_shape=jax.ShapeDtypeStruct((n, d_out), jnp.float32),
    )(part, od, idg, b_src.reshape(1, d_out), b_dst.reshape(1, d_out))
    return out
